# baseline (speedup 1.0000x reference)
import jax
import jax.numpy as jnp
from jax import lax
from jax.experimental import pallas as pl
from jax.experimental.pallas import tpu as pltpu
from jax.experimental.pallas import tpu_sc as plsc

_B, _F, _C = 1024, 26, 1000
_N = _F * _C
_NW = 32
_RPW = _B // _NW
_RCH = 8
_NCH = _RPW // _RCH
_FP = 32
_CB = 4096
_NFB = _N // _CB
_TW = _N % _CB


def _sc_body(colidx_hbm, out_hbm, idx_v, bufs, sems):
    wid = lax.axis_index("s") * 2 + lax.axis_index("c")
    base = wid * _RPW
    ones = jnp.ones((16,), jnp.int32)
    zeros = jnp.zeros((16,), jnp.int32)
    i16 = lax.iota(jnp.int32, 16)
    mask_hi = i16 < (_F - 16)

    for buf, cbw in zip(bufs, (_CB, _CB, _TW)):
        def _zb(j, carry, buf=buf, cbw=cbw):
            r = j // (cbw // 16)
            o = (j % (cbw // 16)) * 16
            buf[r, pl.ds(pl.multiple_of(o, 16), 16)] = zeros
            return carry
        lax.fori_loop(0, (_RCH * cbw) // 16, _zb, None)

    def _scatter(buf, cvecs, c0, cbw, val):
        for r in range(_RCH):
            rvec = jnp.full((16,), r, jnp.int32)
            for g, gm in ((0, None), (1, mask_hi)):
                cvec = cvecs[r][g]
                m = (cvec >= c0) & (cvec < c0 + cbw)
                if gm is not None:
                    m = m & gm
                plsc.store_scatter(buf, [rvec, cvec - c0], val, mask=m)

    def _chunk(ch, carry):
        row0 = base + ch * _RCH
        pltpu.sync_copy(colidx_hbm.at[pl.ds(row0, _RCH)], idx_v)
        cvecs = [[idx_v[r, pl.ds(0, 16)], idx_v[r, pl.ds(16, 16)]]
                 for r in range(_RCH)]
        blocks = [(i * _CB, _CB) for i in range(_NFB)] + [(_NFB * _CB, _TW)]
        inflight = {0: None, 1: None, 2: None}
        for i, (c0, cbw) in enumerate(blocks):
            slot = 2 if cbw == _TW else i % 2
            if inflight[slot] is not None:
                c0p, cp = inflight[slot]
                cp.wait()
                _scatter(bufs[slot], cvecs, c0p, _CB, zeros)
            _scatter(bufs[slot], cvecs, c0, cbw, ones)
            cp = pltpu.async_copy(
                bufs[slot], out_hbm.at[pl.ds(row0, _RCH), pl.ds(c0, cbw)],
                sems[slot])
            inflight[slot] = (c0, cp)
        for slot in (0, 1, 2):
            if inflight[slot] is not None:
                c0p, cp = inflight[slot]
                cp.wait()
                _scatter(bufs[slot], cvecs, c0p,
                         _TW if slot == 2 else _CB, zeros)
        return carry

    lax.fori_loop(0, _NCH, _chunk, None)


def kernel(x):
    colidx = x + jnp.arange(_F, dtype=jnp.int32) * _C
    colidx = jnp.pad(colidx, ((0, 0), (0, _FP - _F)))
    fn = pl.kernel(
        _sc_body,
        out_type=jax.ShapeDtypeStruct((_B, _N), jnp.int32),
        mesh=plsc.VectorSubcoreMesh(core_axis_name="c", subcore_axis_name="s"),
        scratch_types=[
            pltpu.VMEM((_RCH, _FP), jnp.int32),
            (pltpu.VMEM((_RCH, _CB), jnp.int32),
             pltpu.VMEM((_RCH, _CB), jnp.int32),
             pltpu.VMEM((_RCH, _TW), jnp.int32)),
            (pltpu.SemaphoreType.DMA,
             pltpu.SemaphoreType.DMA,
             pltpu.SemaphoreType.DMA),
        ],
        compiler_params=pltpu.CompilerParams(needs_layout_passes=False,
                                             use_tc_tiling_on_sc=True),
    )
    return fn(colidx)


# speedup vs baseline: 4.9106x; 4.9106x over previous
import jax
import jax.numpy as jnp
from jax.experimental import pallas as pl

_B, _F, _C = 1024, 26, 1000
_N = _F * _C


def _body(xT_ref, o_ref):
    cio = jax.lax.broadcasted_iota(jnp.int32, (_C, _B), 0)
    o_ref[...] = (cio == xT_ref[0]).astype(jnp.int32)


def kernel(x):
    xT = x.T.reshape(_F, 1, _B)
    outT = pl.pallas_call(
        _body,
        grid=(_F,),
        in_specs=[pl.BlockSpec((1, 1, _B), lambda i: (i, 0, 0))],
        out_specs=pl.BlockSpec((_C, _B), lambda i: (i, 0)),
        out_shape=jax.ShapeDtypeStruct((_N, _B), jnp.int32),
    )(xT)
    return outT.T
